# baseline (speedup 1.0000x reference)
import functools

import jax
import jax.numpy as jnp
from jax import lax
from jax.experimental import pallas as pl
from jax.experimental.pallas import tpu as pltpu
from jax.experimental.pallas import tpu_sc as plsc

_C = 768
_K = 77
_L = 512
_INT_MIN = -2147483648
_B = 64
_HW = 1024
_B_SC = 16
_NC = 2
_NS = 16
_NW = _NC * _NS
_COLS_PER_W = _HW // _NW
_LANES = 16


def _sortable(s):
    return jnp.where(s < 0, s ^ jnp.int32(0x7FFFFFFF), s)


def _unsortable(p):
    return jnp.where(p < 0, p ^ jnp.int32(0x7FFFFFFF), p)


def _kw_block(dc_ref, x_ref, o_ref):
    xb = x_ref[0]
    scale = jnp.exp(-dc_ref[...])
    boosted = xb * scale

    skey = _sortable(lax.bitcast_convert_type(boosted, jnp.int32))

    def count_ge(cand):
        return jnp.sum((skey >= cand).astype(jnp.int32), axis=0, keepdims=True)

    zero = jnp.zeros((1, xb.shape[1]), jnp.int32)
    p = jnp.where(count_ge(zero) >= _K, zero, jnp.full_like(zero, jnp.int32(_INT_MIN)))
    for bit in range(30, -1, -1):
        cand = p | jnp.int32(1 << bit)
        p = jnp.where(count_ge(cand) >= _K, cand, p)

    thresh = lax.bitcast_convert_type(_unsortable(p), jnp.float32)
    o_ref[0] = jnp.where(boosted < thresh, jnp.zeros_like(xb), xb)


def _mask_block(dc_ref, x_ref, t_ref, prev_ref, o_ref):
    del prev_ref
    xb = x_ref[0]
    scale = jnp.exp(-dc_ref[...])
    boosted = xb * scale
    thresh = t_ref[0]
    o_ref[0] = jnp.where(boosted < thresh, jnp.zeros_like(xb), xb)


def _sc_thresh_body(x_hbm, scale_hbm, out_hbm, xv, kv, sv, tv):
    wid = lax.axis_index("s") * _NC + lax.axis_index("c")
    col_base = wid * _COLS_PER_W

    pltpu.sync_copy(scale_hbm, sv)

    def _per_batch(b, carry):
        for sub in range(_COLS_PER_W // _LANES):
            col0 = col_base + sub * _LANES
            pltpu.sync_copy(x_hbm.at[b, :, pl.ds(col0, _LANES)], xv)

            def _build(c, carry2):
                row = xv[c]
                boosted = row * sv[c]
                s = plsc.bitcast(boosted, jnp.int32)
                kv[c] = jnp.where(s < 0, s ^ jnp.int32(0x7FFFFFFF), s)
                return carry2

            lax.fori_loop(0, _C, _build, 0)

            def _count_ge(cand):
                def _step(c, cnt):
                    return cnt + jnp.where(kv[c] >= cand,
                                           jnp.int32(1), jnp.int32(0))
                return lax.fori_loop(0, _C, _step, jnp.zeros((_LANES,), jnp.int32))

            zero = jnp.zeros((_LANES,), jnp.int32)
            p = jnp.where(_count_ge(zero) >= _K, zero,
                          jnp.full((_LANES,), jnp.int32(_INT_MIN)))
            for bit in range(30, -1, -1):
                cand = p | jnp.int32(1 << bit)
                p = jnp.where(_count_ge(cand) >= _K, cand, p)

            s_t = jnp.where(p < 0, p ^ jnp.int32(0x7FFFFFFF), p)
            tv[...] = plsc.bitcast(s_t, jnp.float32)
            pltpu.sync_copy(tv, out_hbm.at[b, pl.ds(col0, _LANES)])
        return carry

    lax.fori_loop(0, _B_SC, _per_batch, 0)


_sc_thresh = functools.partial(
    pl.kernel,
    out_type=jax.ShapeDtypeStruct((_B_SC, _HW), jnp.float32),
    mesh=plsc.VectorSubcoreMesh(core_axis_name="c", subcore_axis_name="s",
                                num_cores=_NC, num_subcores=_NS),
    compiler_params=pltpu.CompilerParams(use_tc_tiling_on_sc=False,
                                         needs_layout_passes=False),
    scratch_types=[
        pltpu.VMEM((_C, _LANES), jnp.float32),
        pltpu.VMEM((_C, _LANES), jnp.int32),
        pltpu.VMEM((_C, _LANES), jnp.float32),
        pltpu.VMEM((_LANES,), jnp.float32),
    ],
)(_sc_thresh_body)


def kernel(x, duty_cycles):
    B, C, H, W = x.shape
    hw = H * W
    x3 = x.reshape(B, C, hw)
    dc = duty_cycles.reshape(C, 1)

    scale16 = jnp.broadcast_to(jnp.exp(-duty_cycles.reshape(C, 1)), (C, _LANES))
    t_sc = _sc_thresh(x3, scale16)

    out1 = pl.pallas_call(
        _kw_block,
        grid=(B - _B_SC, hw // _L),
        in_specs=[
            pl.BlockSpec((C, 1), lambda b, j: (0, 0)),
            pl.BlockSpec((1, C, _L), lambda b, j: (b + _B_SC, 0, j)),
        ],
        out_specs=pl.BlockSpec((1, C, _L), lambda b, j: (b + _B_SC, 0, j)),
        out_shape=jax.ShapeDtypeStruct((B, C, hw), jnp.float32),
        compiler_params=pltpu.CompilerParams(
            dimension_semantics=("parallel", "parallel"),
        ),
    )(dc, x3)

    out = pl.pallas_call(
        _mask_block,
        grid=(_B_SC, hw // _L),
        in_specs=[
            pl.BlockSpec((C, 1), lambda b, j: (0, 0)),
            pl.BlockSpec((1, C, _L), lambda b, j: (b, 0, j)),
            pl.BlockSpec((1, 1, _L), lambda b, j: (b, 0, j)),
            pl.BlockSpec((1, 8, 128), lambda b, j: (0, 0, 0)),
        ],
        out_specs=pl.BlockSpec((1, C, _L), lambda b, j: (b, 0, j)),
        out_shape=jax.ShapeDtypeStruct((B, C, hw), jnp.float32),
        input_output_aliases={3: 0},
        compiler_params=pltpu.CompilerParams(
            dimension_semantics=("parallel", "parallel"),
        ),
    )(dc, x3, t_sc.reshape(_B_SC, 1, hw), out1)
    return out.reshape(B, C, H, W)


# speedup vs baseline: 3.0473x; 3.0473x over previous
import functools

import jax
import jax.numpy as jnp
from jax import lax
from jax.experimental import pallas as pl
from jax.experimental.pallas import tpu as pltpu
from jax.experimental.pallas import tpu_sc as plsc

_C = 768
_K = 77
_L = 512
_INT_MIN = -2147483648
_B = 64
_HW = 1024
_B_SC = 16
_NC = 2
_NS = 16
_NW = _NC * _NS
_COLS_PER_W = _HW // _NW
_LANES = 16


def _sortable(s):
    return jnp.where(s < 0, s ^ jnp.int32(0x7FFFFFFF), s)


def _unsortable(p):
    return jnp.where(p < 0, p ^ jnp.int32(0x7FFFFFFF), p)


def _kw_block(dc_ref, x_ref, o_ref):
    xb = x_ref[0]
    scale = jnp.exp(-dc_ref[...])
    boosted = xb * scale

    skey = _sortable(lax.bitcast_convert_type(boosted, jnp.int32))

    def count_ge(cand):
        return jnp.sum((skey >= cand).astype(jnp.int32), axis=0, keepdims=True)

    zero = jnp.zeros((1, xb.shape[1]), jnp.int32)
    p = jnp.where(count_ge(zero) >= _K, zero, jnp.full_like(zero, jnp.int32(_INT_MIN)))
    for bit in range(30, -1, -1):
        cand = p | jnp.int32(1 << bit)
        p = jnp.where(count_ge(cand) >= _K, cand, p)

    thresh = lax.bitcast_convert_type(_unsortable(p), jnp.float32)
    o_ref[0] = jnp.where(boosted < thresh, jnp.zeros_like(xb), xb)


def _mask_block(dc_ref, x_ref, t_ref, prev_ref, o_ref):
    del prev_ref
    xb = x_ref[0]
    scale = jnp.exp(-dc_ref[...])
    boosted = xb * scale
    thresh = t_ref[0]
    o_ref[0] = jnp.where(boosted < thresh, jnp.zeros_like(xb), xb)


def _sc_thresh_body(x_hbm, scale_hbm, out_hbm, xv, kv, sv, tv):
    wid = lax.axis_index("s") * _NC + lax.axis_index("c")
    col_base = wid * _COLS_PER_W

    pltpu.sync_copy(scale_hbm, sv)

    def _per_batch(b, carry):
        for sub in range(_COLS_PER_W // _LANES):
            col0 = col_base + sub * _LANES
            pltpu.sync_copy(x_hbm.at[b, :, pl.ds(col0, _LANES)], xv)

            @plsc.parallel_loop(0, _C, step=8, unroll=2)
            def _build(c):
                for i in range(8):
                    row = xv[c + i]
                    boosted = row * sv[c + i]
                    s = plsc.bitcast(boosted, jnp.int32)
                    kv[c + i] = jnp.where(s < 0, s ^ jnp.int32(0x7FFFFFFF), s)

            def _count_ge(cand):
                @plsc.parallel_loop(0, _C, step=16, unroll=2,
                                    carry=jnp.zeros((_LANES,), jnp.int32))
                def _cnt(c, cnt):
                    parts = [jnp.where(kv[c + i] >= cand, jnp.int32(1),
                                       jnp.int32(0)) for i in range(16)]
                    while len(parts) > 1:
                        parts = [a + b for a, b in
                                 zip(parts[0::2], parts[1::2])]
                    return cnt + parts[0]
                return _cnt

            zero = jnp.zeros((_LANES,), jnp.int32)
            p = jnp.where(_count_ge(zero) >= _K, zero,
                          jnp.full((_LANES,), jnp.int32(_INT_MIN)))
            for bit in range(30, -1, -1):
                cand = p | jnp.int32(1 << bit)
                p = jnp.where(_count_ge(cand) >= _K, cand, p)

            s_t = jnp.where(p < 0, p ^ jnp.int32(0x7FFFFFFF), p)
            tv[...] = plsc.bitcast(s_t, jnp.float32)
            pltpu.sync_copy(tv, out_hbm.at[b, pl.ds(col0, _LANES)])
        return carry

    lax.fori_loop(0, _B_SC, _per_batch, 0)


_sc_thresh = functools.partial(
    pl.kernel,
    out_type=jax.ShapeDtypeStruct((_B_SC, _HW), jnp.float32),
    mesh=plsc.VectorSubcoreMesh(core_axis_name="c", subcore_axis_name="s",
                                num_cores=_NC, num_subcores=_NS),
    compiler_params=pltpu.CompilerParams(use_tc_tiling_on_sc=False,
                                         needs_layout_passes=False),
    scratch_types=[
        pltpu.VMEM((_C, _LANES), jnp.float32),
        pltpu.VMEM((_C, _LANES), jnp.int32),
        pltpu.VMEM((_C, _LANES), jnp.float32),
        pltpu.VMEM((_LANES,), jnp.float32),
    ],
)(_sc_thresh_body)


def kernel(x, duty_cycles):
    B, C, H, W = x.shape
    hw = H * W
    x3 = x.reshape(B, C, hw)
    dc = duty_cycles.reshape(C, 1)

    scale16 = jnp.broadcast_to(jnp.exp(-duty_cycles.reshape(C, 1)), (C, _LANES))
    t_sc = _sc_thresh(x3, scale16)

    out1 = pl.pallas_call(
        _kw_block,
        grid=(B - _B_SC, hw // _L),
        in_specs=[
            pl.BlockSpec((C, 1), lambda b, j: (0, 0)),
            pl.BlockSpec((1, C, _L), lambda b, j: (b + _B_SC, 0, j)),
        ],
        out_specs=pl.BlockSpec((1, C, _L), lambda b, j: (b + _B_SC, 0, j)),
        out_shape=jax.ShapeDtypeStruct((B, C, hw), jnp.float32),
        compiler_params=pltpu.CompilerParams(
            dimension_semantics=("parallel", "parallel"),
        ),
    )(dc, x3)

    out = pl.pallas_call(
        _mask_block,
        grid=(_B_SC, hw // _L),
        in_specs=[
            pl.BlockSpec((C, 1), lambda b, j: (0, 0)),
            pl.BlockSpec((1, C, _L), lambda b, j: (b, 0, j)),
            pl.BlockSpec((1, 1, _L), lambda b, j: (b, 0, j)),
            pl.BlockSpec((1, 8, 128), lambda b, j: (0, 0, 0)),
        ],
        out_specs=pl.BlockSpec((1, C, _L), lambda b, j: (b, 0, j)),
        out_shape=jax.ShapeDtypeStruct((B, C, hw), jnp.float32),
        input_output_aliases={3: 0},
        compiler_params=pltpu.CompilerParams(
            dimension_semantics=("parallel", "parallel"),
        ),
    )(dc, x3, t_sc.reshape(_B_SC, 1, hw), out1)
    return out.reshape(B, C, H, W)


# speedup vs baseline: 3.0816x; 1.0112x over previous
import functools

import jax
import jax.numpy as jnp
from jax import lax
from jax.experimental import pallas as pl
from jax.experimental.pallas import tpu as pltpu
from jax.experimental.pallas import tpu_sc as plsc

_C = 768
_K = 77
_L = 512
_INT_MIN = -2147483648
_B = 64
_HW = 1024
_B_SC = 16
_NC = 2
_NS = 16
_NW = _NC * _NS
_COLS_PER_W = _HW // _NW
_LANES = 16


def _sortable(s):
    return jnp.where(s < 0, s ^ jnp.int32(0x7FFFFFFF), s)


def _unsortable(p):
    return jnp.where(p < 0, p ^ jnp.int32(0x7FFFFFFF), p)


def _kw_block(dc_ref, x_ref, o_ref):
    xb = x_ref[0]
    scale = jnp.exp(-dc_ref[...])
    boosted = xb * scale

    skey = _sortable(lax.bitcast_convert_type(boosted, jnp.int32))

    def count_ge(cand):
        return jnp.sum((skey >= cand).astype(jnp.int32), axis=0, keepdims=True)

    zero = jnp.zeros((1, xb.shape[1]), jnp.int32)
    p = jnp.where(count_ge(zero) >= _K, zero, jnp.full_like(zero, jnp.int32(_INT_MIN)))
    for bit in range(30, -1, -1):
        cand = p | jnp.int32(1 << bit)
        p = jnp.where(count_ge(cand) >= _K, cand, p)

    thresh = lax.bitcast_convert_type(_unsortable(p), jnp.float32)
    o_ref[0] = jnp.where(boosted < thresh, jnp.zeros_like(xb), xb)


def _mask_block(dc_ref, x_ref, t_ref, prev_ref, o_ref):
    del prev_ref
    xb = x_ref[0]
    scale = jnp.exp(-dc_ref[...])
    boosted = xb * scale
    thresh = t_ref[0]
    o_ref[0] = jnp.where(boosted < thresh, jnp.zeros_like(xb), xb)


def _sc_thresh_body(x_hbm, scale_hbm, out_hbm, xv, kv, sv, hv, tv):
    wid = lax.axis_index("s") * _NC + lax.axis_index("c")
    col_base = wid * _COLS_PER_W

    pltpu.sync_copy(scale_hbm, sv)
    lanes = lax.iota(jnp.int32, _LANES)

    def _per_batch(b, carry):
        for sub in range(_COLS_PER_W // _LANES):
            col0 = col_base + sub * _LANES
            pltpu.sync_copy(x_hbm.at[b, :, pl.ds(col0, _LANES)], xv)

            @plsc.parallel_loop(0, _C, step=8, unroll=2)
            def _build(c):
                for j in range(8):
                    row = xv[c + j]
                    boosted = row * sv[c + j]
                    s = plsc.bitcast(boosted, jnp.int32)
                    key = jnp.where(s < 0, s ^ jnp.int32(0x7FFFFFFF), s)
                    kv[c + j] = key ^ jnp.int32(_INT_MIN)

            prefix = jnp.zeros((_LANES,), jnp.int32)
            krem = jnp.full((_LANES,), jnp.int32(_K))
            ones16 = jnp.ones((_LANES,), jnp.int32)
            for pno in range(4):
                shift = 24 - 8 * pno

                @plsc.parallel_loop(0, 256, step=8)
                def _clear(d):
                    for j in range(8):
                        hv[d + j] = jnp.zeros((_LANES,), jnp.int32)

                @plsc.parallel_loop(0, _C, step=8)
                def _scatter(c):
                    for j in range(8):
                        key = kv[c + j]
                        digit = lax.shift_right_logical(key, shift) & jnp.int32(0xFF)
                        if pno == 0:
                            elig = None
                        else:
                            hi = lax.shift_right_logical(key, shift + 8)
                            elig = hi == prefix
                        plsc.addupdate_scatter(hv, [digit, lanes], ones16,
                                               mask=elig)

                def _scan(i, st):
                    acc, dsel, above, found = st
                    d = jnp.int32(255) - i
                    h = hv[d]
                    acc2 = acc + h
                    cross = jnp.logical_and(jnp.logical_not(found),
                                            acc2 >= krem)
                    dsel = jnp.where(cross, d, dsel)
                    above = jnp.where(cross, acc, above)
                    found = jnp.logical_or(found, cross)
                    return acc2, dsel, above, found

                zero = jnp.zeros((_LANES,), jnp.int32)
                _, dsel, above, _ = lax.fori_loop(
                    0, 256, _scan,
                    (zero, zero, zero, jnp.zeros((_LANES,), jnp.bool_)))
                prefix = (prefix << 8) | dsel
                krem = krem - above

            skey = prefix ^ jnp.int32(_INT_MIN)
            s_t = jnp.where(skey < 0, skey ^ jnp.int32(0x7FFFFFFF), skey)
            tv[...] = plsc.bitcast(s_t, jnp.float32)
            pltpu.sync_copy(tv, out_hbm.at[b, pl.ds(col0, _LANES)])
        return carry

    lax.fori_loop(0, _B_SC, _per_batch, 0)


_sc_thresh = functools.partial(
    pl.kernel,
    out_type=jax.ShapeDtypeStruct((_B_SC, _HW), jnp.float32),
    mesh=plsc.VectorSubcoreMesh(core_axis_name="c", subcore_axis_name="s",
                                num_cores=_NC, num_subcores=_NS),
    compiler_params=pltpu.CompilerParams(use_tc_tiling_on_sc=False,
                                         needs_layout_passes=False),
    scratch_types=[
        pltpu.VMEM((_C, _LANES), jnp.float32),
        pltpu.VMEM((_C, _LANES), jnp.int32),
        pltpu.VMEM((_C, _LANES), jnp.float32),
        pltpu.VMEM((256, _LANES), jnp.int32),
        pltpu.VMEM((_LANES,), jnp.float32),
    ],
)(_sc_thresh_body)


def kernel(x, duty_cycles):
    B, C, H, W = x.shape
    hw = H * W
    x3 = x.reshape(B, C, hw)
    dc = duty_cycles.reshape(C, 1)

    scale16 = jnp.broadcast_to(jnp.exp(-duty_cycles.reshape(C, 1)), (C, _LANES))
    t_sc = _sc_thresh(x3, scale16)

    out1 = pl.pallas_call(
        _kw_block,
        grid=(B - _B_SC, hw // _L),
        in_specs=[
            pl.BlockSpec((C, 1), lambda b, j: (0, 0)),
            pl.BlockSpec((1, C, _L), lambda b, j: (b + _B_SC, 0, j)),
        ],
        out_specs=pl.BlockSpec((1, C, _L), lambda b, j: (b + _B_SC, 0, j)),
        out_shape=jax.ShapeDtypeStruct((B, C, hw), jnp.float32),
        compiler_params=pltpu.CompilerParams(
            dimension_semantics=("parallel", "parallel"),
        ),
    )(dc, x3)

    out = pl.pallas_call(
        _mask_block,
        grid=(_B_SC, hw // _L),
        in_specs=[
            pl.BlockSpec((C, 1), lambda b, j: (0, 0)),
            pl.BlockSpec((1, C, _L), lambda b, j: (b, 0, j)),
            pl.BlockSpec((1, 1, _L), lambda b, j: (b, 0, j)),
            pl.BlockSpec((1, 8, 128), lambda b, j: (0, 0, 0)),
        ],
        out_specs=pl.BlockSpec((1, C, _L), lambda b, j: (b, 0, j)),
        out_shape=jax.ShapeDtypeStruct((B, C, hw), jnp.float32),
        input_output_aliases={3: 0},
        compiler_params=pltpu.CompilerParams(
            dimension_semantics=("parallel", "parallel"),
        ),
    )(dc, x3, t_sc.reshape(_B_SC, 1, hw), out1)
    return out.reshape(B, C, H, W)
